# Initial kernel scaffold; baseline (speedup 1.0000x reference)
import jax
import jax.numpy as jnp
from jax.experimental import pallas as pl


def kernel(tag_ids, table, W1, b1, W2, b2, W3, b3):
    raise NotImplementedError("write your pallas kernel here")


# speedup vs baseline: 1.5340x; 1.5340x over previous
import functools

import jax
import jax.numpy as jnp
from jax import lax
from jax.experimental import pallas as pl
from jax.experimental.pallas import tpu as pltpu
from jax.experimental.pallas import tpu_sc as plsc

D = 64
L = 200
B = 16384
IDXW = 100
RB = 2
STEP_IROWS = 2 * RB
NCHUNK = D // 16

_info = plsc.get_sparse_core_info()
NC, NS = _info.num_cores, _info.num_subcores
NW = NC * NS


def _sc_partial_sums(ids2d, table):
    irows_per_w = ids2d.shape[0] // NW
    nsteps = irows_per_w // STEP_IROWS
    mesh = plsc.VectorSubcoreMesh(core_axis_name="c", subcore_axis_name="s")

    @functools.partial(
        pl.kernel,
        mesh=mesh,
        out_type=jax.ShapeDtypeStruct((NW, L, D), jnp.float32),
        compiler_params=pltpu.CompilerParams(use_tc_tiling_on_sc=False),
        scratch_types=[
            pltpu.VMEM((STEP_IROWS, IDXW), jnp.int32),
            pltpu.VMEM((STEP_IROWS, IDXW), jnp.int32),
            pltpu.VMEM((RB * L, D), jnp.float32),
            pltpu.VMEM((RB * L, D), jnp.float32),
            pltpu.VMEM((L, D), jnp.float32),
            pltpu.SemaphoreType.DMA,
            pltpu.SemaphoreType.DMA,
        ],
    )
    def k(ids_hbm, table_hbm, out_hbm, idx0, idx1, buf0, buf1, acc, sem0, sem1):
        wid = lax.axis_index("s") * NC + lax.axis_index("c")
        base_irow = wid * irows_per_w
        idxbufs = (idx0, idx1)
        bufs = (buf0, buf1)
        sems = (sem0, sem1)

        def zero_body(l, _):
            for c in range(NCHUNK):
                acc[l, pl.ds(c * 16, 16)] = jnp.zeros((16,), jnp.float32)
            return 0
        lax.fori_loop(0, L, zero_body, 0)

        def fire(g, slot):
            irow = base_irow + g * STEP_IROWS
            pltpu.sync_copy(ids_hbm.at[pl.ds(irow, STEP_IROWS)], idxbufs[slot])
            for j in range(STEP_IROWS):
                pltpu.async_copy(
                    table_hbm.at[idxbufs[slot].at[j]],
                    bufs[slot].at[pl.ds(j * IDXW, IDXW)],
                    sems[slot],
                )

        def drain(slot):
            pltpu.make_async_copy(
                table_hbm.at[pl.ds(0, RB * L)], bufs[slot], sems[slot]
            ).wait()

        def accumulate(slot):
            buf = bufs[slot]
            def body(l, _):
                for r in range(RB):
                    row = r * L + l
                    for c in range(NCHUNK):
                        plsc.addupdate(
                            acc.at[l, pl.ds(c * 16, 16)],
                            buf[row, pl.ds(c * 16, 16)],
                        )
                return 0
            lax.fori_loop(0, L, body, 0)

        fire(0, 0)

        def main_body(g2, _):
            g = g2 * 2
            fire(g + 1, 1)
            drain(0)
            accumulate(0)
            fire(g + 2, 0)
            drain(1)
            accumulate(1)
            return 0
        lax.fori_loop(0, nsteps // 2 - 1, main_body, 0)

        fire(nsteps - 1, 1)
        drain(0)
        accumulate(0)
        drain(1)
        accumulate(1)

        pltpu.sync_copy(acc, out_hbm.at[wid])

    return k(ids2d, table)


def _mlp(partials, W1, b1, W2, b2, W3, b3):
    def body(p_ref, w1_ref, b1_ref, w2_ref, b2_ref, w3_ref, b3_ref, o_ref):
        s = jnp.sum(p_ref[...], axis=0) * (1.0 / B)
        h = jnp.maximum(
            jnp.dot(s, w1_ref[...], preferred_element_type=jnp.float32)
            + b1_ref[...], 0.0)
        h = jnp.maximum(
            jnp.dot(h, w2_ref[...], preferred_element_type=jnp.float32)
            + b2_ref[...], 0.0)
        o_ref[...] = (
            jnp.dot(h, w3_ref[...], preferred_element_type=jnp.float32)
            + b3_ref[...])

    return pl.pallas_call(
        body,
        out_shape=jax.ShapeDtypeStruct((L, D), jnp.float32),
    )(partials, W1, b1.reshape(1, D), W2, b2.reshape(1, D), W3,
      b3.reshape(1, D))


def kernel(tag_ids, table, W1, b1, W2, b2, W3, b3):
    ids2d = tag_ids.astype(jnp.int32).reshape(-1, IDXW)
    partials = _sc_partial_sums(ids2d, table)
    return _mlp(partials, W1, b1, W2, b2, W3, b3)


# speedup vs baseline: 1.9722x; 1.2857x over previous
import functools

import jax
import jax.numpy as jnp
from jax import lax
from jax.experimental import pallas as pl
from jax.experimental.pallas import tpu as pltpu
from jax.experimental.pallas import tpu_sc as plsc

D = 64
L = 200
B = 16384
IDXW = 100
RB = 4
STEP_IROWS = 2 * RB
NCHUNK = D // 16

_info = plsc.get_sparse_core_info()
NC, NS = _info.num_cores, _info.num_subcores
NW = NC * NS


def _sc_partial_sums(ids2d, table):
    irows_per_w = ids2d.shape[0] // NW
    nsteps = irows_per_w // STEP_IROWS
    mesh = plsc.VectorSubcoreMesh(core_axis_name="c", subcore_axis_name="s")

    @functools.partial(
        pl.kernel,
        mesh=mesh,
        out_type=jax.ShapeDtypeStruct((NW, L, D), jnp.float32),
        compiler_params=pltpu.CompilerParams(use_tc_tiling_on_sc=False),
        scratch_types=[
            pltpu.VMEM((STEP_IROWS, IDXW), jnp.int32),
            pltpu.VMEM((STEP_IROWS, IDXW), jnp.int32),
            pltpu.VMEM((RB * L, D), jnp.float32),
            pltpu.VMEM((RB * L, D), jnp.float32),
            pltpu.VMEM((L, D), jnp.float32),
            pltpu.SemaphoreType.DMA,
            pltpu.SemaphoreType.DMA,
            pltpu.SemaphoreType.DMA,
        ],
    )
    def k(ids_hbm, table_hbm, out_hbm, idx0, idx1, buf0, buf1, acc,
          sem0, sem1, isem):
        wid = lax.axis_index("s") * NC + lax.axis_index("c")
        base_irow = wid * irows_per_w
        idxbufs = (idx0, idx1)
        bufs = (buf0, buf1)
        sems = (sem0, sem1)

        def zero_body(l, _):
            for c in range(NCHUNK):
                acc[l, pl.ds(c * 16, 16)] = jnp.zeros((16,), jnp.float32)
            return 0
        lax.fori_loop(0, L, zero_body, 0)

        def idx_fetch(g, slot):
            irow = base_irow + g * STEP_IROWS
            pltpu.async_copy(ids_hbm.at[pl.ds(irow, STEP_IROWS)],
                             idxbufs[slot], isem)

        def idx_wait(slot):
            pltpu.make_async_copy(ids_hbm.at[pl.ds(0, STEP_IROWS)],
                                  idxbufs[slot], isem).wait()

        def fire(slot):
            for j in range(STEP_IROWS):
                pltpu.async_copy(
                    table_hbm.at[idxbufs[slot].at[j]],
                    bufs[slot].at[pl.ds(j * IDXW, IDXW)],
                    sems[slot],
                )

        def drain(slot):
            pltpu.make_async_copy(
                table_hbm.at[pl.ds(0, RB * L)], bufs[slot], sems[slot]
            ).wait()

        def accumulate(slot):
            buf = bufs[slot]
            def body(l, _):
                for c in range(NCHUNK):
                    sl = pl.ds(c * 16, 16)
                    v = buf[l, sl]
                    for r in range(1, RB):
                        v = v + buf[r * L + l, sl]
                    plsc.addupdate(acc.at[l, sl], v)
                return 0
            lax.fori_loop(0, L, body, 0)

        idx_fetch(0, 0)
        idx_wait(0)
        fire(0)
        idx_fetch(1, 1)

        def phase(g, slot, nslot, fetch_ahead):
            idx_wait(nslot)
            fire(nslot)
            drain(slot)
            if fetch_ahead:
                idx_fetch(g + 2, slot)
            accumulate(slot)

        def main_body(g2, _):
            g = g2 * 2
            phase(g, 0, 1, True)
            phase(g + 1, 1, 0, True)
            return 0
        lax.fori_loop(0, nsteps // 2 - 1, main_body, 0)

        phase(nsteps - 2, 0, 1, False)
        drain(1)
        accumulate(1)

        pltpu.sync_copy(acc, out_hbm.at[wid])

    return k(ids2d, table)


def _mlp(partials, W1, b1, W2, b2, W3, b3):
    def body(p_ref, w1_ref, b1_ref, w2_ref, b2_ref, w3_ref, b3_ref, o_ref):
        s = jnp.sum(p_ref[...], axis=0) * (1.0 / B)
        h = jnp.maximum(
            jnp.dot(s, w1_ref[...], preferred_element_type=jnp.float32)
            + b1_ref[...], 0.0)
        h = jnp.maximum(
            jnp.dot(h, w2_ref[...], preferred_element_type=jnp.float32)
            + b2_ref[...], 0.0)
        o_ref[...] = (
            jnp.dot(h, w3_ref[...], preferred_element_type=jnp.float32)
            + b3_ref[...])

    return pl.pallas_call(
        body,
        out_shape=jax.ShapeDtypeStruct((L, D), jnp.float32),
    )(partials, W1, b1.reshape(1, D), W2, b2.reshape(1, D), W3,
      b3.reshape(1, D))


def kernel(tag_ids, table, W1, b1, W2, b2, W3, b3):
    ids2d = tag_ids.astype(jnp.int32).reshape(-1, IDXW)
    partials = _sc_partial_sums(ids2d, table)
    return _mlp(partials, W1, b1, W2, b2, W3, b3)
